# Initial kernel scaffold; baseline (speedup 1.0000x reference)
import jax
import jax.numpy as jnp
from jax.experimental import pallas as pl


def kernel(x, edge_index, edge_weight, W1, b1, W2, b2):
    raise NotImplementedError("write your pallas kernel here")


# speedup vs baseline: 2.5914x; 2.5914x over previous
import functools

import jax
import jax.numpy as jnp
from jax import lax
from jax.experimental import pallas as pl
from jax.experimental.pallas import tpu as pltpu
from jax.experimental.pallas import tpu_sc as plsc

N = 10000
D = 128
E = 320000
NC = 2
NS = 16
NW = NC * NS
CHUNK = 128
CPT = 80
E_PAD = NW * CPT * CHUNK
N_PAD = 10240
ROWS_PT = N_PAD // NS


def _bcast16(vec16, e):
    idx = jnp.full((16, 1), e, dtype=jnp.int32)
    dn = lax.GatherDimensionNumbers(
        offset_dims=(), collapsed_slice_dims=(0,), start_index_map=(0,))
    return lax.gather(vec16, idx, dn, (1,),
                      mode=lax.GatherScatterMode.PROMISE_IN_BOUNDS)


def _spmm_body(x_hbm, col_hbm, row_hbm, w_hbm, zero_hbm, out_hbm,
               col_v, row_v, w_v, rows_v, acc, sem):
    c = lax.axis_index("c")
    s = lax.axis_index("s")
    wid = s * NC + c

    pltpu.sync_copy(col_hbm.at[wid], col_v)
    pltpu.sync_copy(row_hbm.at[wid], row_v)
    pltpu.sync_copy(w_hbm.at[wid], w_v)
    pltpu.sync_copy(zero_hbm.at[pl.ds(s * ROWS_PT, ROWS_PT)],
                    acc.at[pl.ds(s * ROWS_PT, ROWS_PT)])
    plsc.subcore_barrier()

    def chunk_body(j, carry):
        pltpu.async_copy(x_hbm.at[col_v.at[j]], rows_v, sem).wait()

        def group_body(g, carry2):
            w16 = w_v[j, pl.ds(g * 16, 16)]
            for e in range(16):
                wb = _bcast16(w16, e)
                eid = g * 16 + e
                for k in range(D // 16):
                    sl = (eid, pl.ds(k * 16, 16))
                    rows_v[sl] = rows_v[sl] * wb
            return carry2

        lax.fori_loop(0, CHUNK // 16, group_body, 0)

        pltpu.sync_copy(rows_v, acc.at[row_v.at[j]], add=True)
        return carry

    lax.fori_loop(0, CPT, chunk_body, 0)

    plsc.subcore_barrier()
    pltpu.sync_copy(acc.at[pl.ds(s * ROWS_PT, ROWS_PT)],
                    out_hbm.at[c, pl.ds(s * ROWS_PT, ROWS_PT)])


_spmm_sc = functools.partial(
    pl.kernel,
    mesh=plsc.VectorSubcoreMesh(core_axis_name="c", subcore_axis_name="s"),
    out_type=jax.ShapeDtypeStruct((NC, N_PAD, D), jnp.float32),
    scratch_types=[
        pltpu.VMEM((CPT, CHUNK), jnp.int32),
        pltpu.VMEM((CPT, CHUNK), jnp.int32),
        pltpu.VMEM((CPT, CHUNK), jnp.float32),
        pltpu.VMEM((CHUNK, D), jnp.float32),
        pltpu.VMEM_SHARED((N_PAD, D), jnp.float32),
        pltpu.SemaphoreType.DMA,
    ],
)(_spmm_body)


_BLK = 1000
_GRID = N // _BLK


def _add2_body(p_ref, o_ref):
    o_ref[...] = p_ref[0] + p_ref[1]


_add2 = pl.pallas_call(
    _add2_body,
    grid=(_GRID,),
    in_specs=[pl.BlockSpec((NC, _BLK, D), lambda i: (0, i, 0))],
    out_specs=pl.BlockSpec((_BLK, D), lambda i: (i, 0)),
    out_shape=jax.ShapeDtypeStruct((N, D), jnp.float32),
)


def _linear_body(p_ref, a_ref, b_ref, wT_ref, bias_ref, o_ref, *, act):
    s2 = p_ref[0] + p_ref[1]
    wT = wT_ref[...]
    acc = jnp.dot(a_ref[...], wT[0:D], preferred_element_type=jnp.float32)
    acc = acc + jnp.dot(b_ref[...], wT[D:2 * D],
                        preferred_element_type=jnp.float32)
    acc = acc + jnp.dot(s2, wT[2 * D:3 * D],
                        preferred_element_type=jnp.float32)
    acc = acc + bias_ref[...]
    if act:
        acc = jnp.where(acc > 0, acc, 0.01 * acc)
    o_ref[...] = acc


def _make_linear(act):
    return pl.pallas_call(
        functools.partial(_linear_body, act=act),
        grid=(_GRID,),
        in_specs=[
            pl.BlockSpec((NC, _BLK, D), lambda i: (0, i, 0)),
            pl.BlockSpec((_BLK, D), lambda i: (i, 0)),
            pl.BlockSpec((_BLK, D), lambda i: (i, 0)),
            pl.BlockSpec((3 * D, D), lambda i: (0, 0)),
            pl.BlockSpec((1, D), lambda i: (0, 0)),
        ],
        out_specs=pl.BlockSpec((_BLK, D), lambda i: (i, 0)),
        out_shape=jax.ShapeDtypeStruct((N, D), jnp.float32),
    )


_linear_act = _make_linear(True)
_linear_noact = _make_linear(False)


def kernel(x, edge_index, edge_weight, W1, b1, W2, b2):
    row = edge_index[0]
    col = edge_index[1]
    pad = E_PAD - E
    colp = jnp.concatenate([col, jnp.zeros((pad,), jnp.int32)])
    rowp = jnp.concatenate([row, jnp.zeros((pad,), jnp.int32)])
    wp = jnp.concatenate([edge_weight, jnp.zeros((pad,), jnp.float32)])
    col3 = colp.reshape(NW, CPT, CHUNK)
    row3 = rowp.reshape(NW, CPT, CHUNK)
    w3 = wp.reshape(NW, CPT, CHUNK)
    zeros = jnp.zeros((N_PAD, D), jnp.float32)
    W1T = W1.T
    W2T = W2.T
    b1r = b1.reshape(1, D)
    b2r = b2.reshape(1, D)

    p1 = _spmm_sc(x, col3, row3, w3, zeros)
    s1 = _add2(p1)
    p2 = _spmm_sc(s1, col3, row3, w3, zeros)
    h = _linear_act(p2, x, s1, W1T, b1r)
    p3 = _spmm_sc(h, col3, row3, w3, zeros)
    t1 = _add2(p3)
    p4 = _spmm_sc(t1, col3, row3, w3, zeros)
    out = _linear_noact(p4, h, t1, W2T, b2r)
    return out
